# baseline (speedup 1.0000x reference)
import functools

import jax
import jax.numpy as jnp
from jax import lax
from jax.experimental import pallas as pl
from jax.experimental.pallas import tpu as pltpu
from jax.experimental.pallas import tpu_sc as plsc

_N_STATES = 1000000
_STAGE_SEG = 10000

_PER_SC = 1638400
_N_SP = 11
_SP_LEN = 120320
_HBM_LEN = 62976
_N_CHUNKS = 8
_SP_CHUNK = _SP_LEN // _N_CHUNKS
_HBM_CHUNK = _HBM_LEN // _N_CHUNKS
_BUF = _SP_CHUNK


@functools.partial(jax.jit, static_argnames=("total",))
def _sc_gather(table, flat_idx, total):
    mesh = plsc.VectorSubcoreMesh(core_axis_name="c", subcore_axis_name="s")

    @functools.partial(
        pl.kernel,
        mesh=mesh,
        out_type=jax.ShapeDtypeStruct((total,), jnp.float32),
        scratch_types=[
            pltpu.VMEM((_BUF,), jnp.int32),
            pltpu.VMEM((_BUF,), jnp.int32),
            pltpu.VMEM((_BUF,), jnp.float32),
            pltpu.VMEM((_BUF,), jnp.float32),
            pltpu.VMEM_SHARED((_N_STATES,), jnp.float32),
            pltpu.SemaphoreType.DMA,
            pltpu.SemaphoreType.DMA,
            pltpu.SemaphoreType.DMA,
            pltpu.SemaphoreType.DMA,
            pltpu.SemaphoreType.DMA,
        ],
    )
    def k(table_hbm, idx_hbm, out_hbm, idx_v0, idx_v1, rows_v0, rows_v1,
          table_sp, sem_i0, sem_i1, sem_g, sem_o0, sem_o1):
        idx_v = (idx_v0, idx_v1)
        rows_v = (rows_v0, rows_v1)
        sem_i = (sem_i0, sem_i1)
        sem_o = (sem_o0, sem_o1)
        sid = lax.axis_index("s")
        cid = lax.axis_index("c")
        base_sc = cid * _PER_SC

        n_pieces = _N_STATES // _STAGE_SEG
        n_rounds = -(-n_pieces // 16)
        for p in range(n_rounds):
            piece = p * 16 + sid

            @pl.when(piece < n_pieces)
            def _stage():
                seg = pl.ds(piece * _STAGE_SEG, _STAGE_SEG)
                stage_v = rows_v0.at[pl.ds(0, _STAGE_SEG)]
                pltpu.sync_copy(table_hbm.at[seg], stage_v)
                pltpu.sync_copy(stage_v, table_sp.at[seg])

        plsc.subcore_barrier()

        def pipeline(off0, chunk, table_ref):
            def idx_load(i):
                return pltpu.async_copy(
                    idx_hbm.at[pl.ds(off0 + i * chunk, chunk)],
                    idx_v[i % 2].at[pl.ds(0, chunk)],
                    sem_i[i % 2],
                )

            loads = {0: idx_load(0), 1: idx_load(1)}
            stores = {}
            for i in range(_N_CHUNKS):
                b = i % 2
                loads[i].wait()
                if i - 2 in stores:
                    stores[i - 2].wait()
                pltpu.async_copy(
                    table_ref.at[idx_v[b].at[pl.ds(0, chunk)]],
                    rows_v[b].at[pl.ds(0, chunk)],
                    sem_g,
                ).wait()
                if i + 2 < _N_CHUNKS:
                    loads[i + 2] = idx_load(i + 2)
                stores[i] = pltpu.async_copy(
                    rows_v[b].at[pl.ds(0, chunk)],
                    out_hbm.at[pl.ds(off0 + i * chunk, chunk)],
                    sem_o[b],
                )
            stores[_N_CHUNKS - 2].wait()
            stores[_N_CHUNKS - 1].wait()

        @pl.when(sid < _N_SP)
        def _sp_path():
            pipeline(base_sc + sid * _SP_LEN, _SP_CHUNK, table_sp)

        @pl.when(sid >= _N_SP)
        def _hbm_path():
            off0 = base_sc + _N_SP * _SP_LEN + (sid - _N_SP) * _HBM_LEN
            pipeline(off0, _HBM_CHUNK, table_hbm)

    return k(table, flat_idx)


def kernel(rew_matrix, state):
    flat = state.reshape(-1)
    out = _sc_gather(rew_matrix, flat, flat.shape[0])
    return out.reshape(state.shape)


# speedup vs baseline: 1.3203x; 1.3203x over previous
import functools

import jax
import jax.numpy as jnp
from jax import lax
from jax.experimental import pallas as pl
from jax.experimental.pallas import tpu as pltpu
from jax.experimental.pallas import tpu_sc as plsc

_N_STATES = 1000000
_STAGE_SEG = 10000

_PER_SC = 1638400
_N_SP = 16
_SP_LEN = 102400
_HBM_LEN = 0
_N_CHUNKS = 8
_SP_CHUNK = _SP_LEN // _N_CHUNKS
_BUF = _SP_CHUNK


@functools.partial(jax.jit, static_argnames=("total",))
def _sc_gather(table, flat_idx, total):
    mesh = plsc.VectorSubcoreMesh(core_axis_name="c", subcore_axis_name="s")

    @functools.partial(
        pl.kernel,
        mesh=mesh,
        out_type=jax.ShapeDtypeStruct((total,), jnp.float32),
        scratch_types=[
            pltpu.VMEM((_BUF,), jnp.int32),
            pltpu.VMEM((_BUF,), jnp.int32),
            pltpu.VMEM((_BUF,), jnp.float32),
            pltpu.VMEM((_BUF,), jnp.float32),
            pltpu.VMEM_SHARED((_N_STATES,), jnp.float32),
            pltpu.SemaphoreType.DMA,
            pltpu.SemaphoreType.DMA,
            pltpu.SemaphoreType.DMA,
            pltpu.SemaphoreType.DMA,
            pltpu.SemaphoreType.DMA,
        ],
    )
    def k(table_hbm, idx_hbm, out_hbm, idx_v0, idx_v1, rows_v0, rows_v1,
          table_sp, sem_i0, sem_i1, sem_g, sem_o0, sem_o1):
        idx_v = (idx_v0, idx_v1)
        rows_v = (rows_v0, rows_v1)
        sem_i = (sem_i0, sem_i1)
        sem_o = (sem_o0, sem_o1)
        sid = lax.axis_index("s")
        cid = lax.axis_index("c")
        base_sc = cid * _PER_SC

        n_pieces = _N_STATES // _STAGE_SEG
        n_rounds = -(-n_pieces // 16)
        for p in range(n_rounds):
            piece = p * 16 + sid

            @pl.when(piece < n_pieces)
            def _stage():
                seg = pl.ds(piece * _STAGE_SEG, _STAGE_SEG)
                stage_v = rows_v0.at[pl.ds(0, _STAGE_SEG)]
                pltpu.sync_copy(table_hbm.at[seg], stage_v)
                pltpu.sync_copy(stage_v, table_sp.at[seg])

        plsc.subcore_barrier()

        def pipeline(off0, chunk, table_ref):
            def idx_load(i):
                return pltpu.async_copy(
                    idx_hbm.at[pl.ds(off0 + i * chunk, chunk)],
                    idx_v[i % 2].at[pl.ds(0, chunk)],
                    sem_i[i % 2],
                )

            loads = {0: idx_load(0), 1: idx_load(1)}
            stores = {}
            for i in range(_N_CHUNKS):
                b = i % 2
                loads[i].wait()
                if i - 2 in stores:
                    stores[i - 2].wait()
                pltpu.async_copy(
                    table_ref.at[idx_v[b].at[pl.ds(0, chunk)]],
                    rows_v[b].at[pl.ds(0, chunk)],
                    sem_g,
                ).wait()
                if i + 2 < _N_CHUNKS:
                    loads[i + 2] = idx_load(i + 2)
                stores[i] = pltpu.async_copy(
                    rows_v[b].at[pl.ds(0, chunk)],
                    out_hbm.at[pl.ds(off0 + i * chunk, chunk)],
                    sem_o[b],
                )
            stores[_N_CHUNKS - 2].wait()
            stores[_N_CHUNKS - 1].wait()

        if _N_SP == 16:
            pipeline(base_sc + sid * _SP_LEN, _SP_CHUNK, table_sp)
        else:
            @pl.when(sid < _N_SP)
            def _sp_path():
                pipeline(base_sc + sid * _SP_LEN, _SP_CHUNK, table_sp)

            @pl.when(sid >= _N_SP)
            def _hbm_path():
                off0 = base_sc + _N_SP * _SP_LEN + (sid - _N_SP) * _HBM_LEN
                pipeline(off0, _HBM_CHUNK, table_hbm)

    return k(table, flat_idx)


def kernel(rew_matrix, state):
    flat = state.reshape(-1)
    out = _sc_gather(rew_matrix, flat, flat.shape[0])
    return out.reshape(state.shape)
